# Initial kernel scaffold; baseline (speedup 1.0000x reference)
import jax
import jax.numpy as jnp
from jax.experimental import pallas as pl


def kernel(s):
    raise NotImplementedError("write your pallas kernel here")


# speedup vs baseline: 91.5397x; 91.5397x over previous
import numpy as np
import jax
import jax.numpy as jnp
from jax.experimental import pallas as pl

NEURONS_C = 32768
K_C = 64
ROWS_PER_BLOCK = 8

MIN32 = np.int32(-2**31)


def _topk_mask_block(s_ref, o_ref):
    x = s_ref[...]
    i = jax.lax.bitcast_convert_type(x, jnp.int32)
    key = i ^ jax.lax.shift_right_arithmetic(i, 31) & np.int32(0x7FFFFFFF)
    u = jnp.zeros((ROWS_PER_BLOCK, 1), jnp.int32)
    for b in range(31, -1, -1):
        bit = MIN32 if b == 31 else np.int32(1 << b)
        cand = u | bit
        thr_s = cand ^ MIN32
        cnt = jnp.sum((key >= thr_s).astype(jnp.int32), axis=-1, keepdims=True)
        u = jnp.where(cnt >= K_C, cand, u)
    thr = u ^ MIN32

    gt = key > thr
    eq = key == thr
    n_gt = jnp.sum(gt.astype(jnp.int32), axis=-1, keepdims=True)
    n_eq = jnp.sum(eq.astype(jnp.int32), axis=-1, keepdims=True)
    need = K_C - n_gt

    any_surplus = jnp.any(n_eq != need)

    @pl.when(jnp.logical_not(any_surplus))
    def _simple():
        o_ref[...] = jnp.where(key >= thr, x, 0.0)

    @pl.when(any_surplus)
    def _tie_break():
        idx = jax.lax.broadcasted_iota(jnp.int32, x.shape, 1)
        j = jnp.zeros((ROWS_PER_BLOCK, 1), jnp.int32)
        for b in range(14, -1, -1):
            cand = j | np.int32(1 << b)
            c = jnp.sum((eq & (idx >= cand)).astype(jnp.int32), axis=-1,
                        keepdims=True)
            j = jnp.where(c >= need, cand, j)
        j = jnp.where(n_eq == need, 0, j)
        o_ref[...] = jnp.where(gt | (eq & (idx >= j)), x, 0.0)


def kernel(s):
    batch, neurons = s.shape
    grid = batch // ROWS_PER_BLOCK
    return pl.pallas_call(
        _topk_mask_block,
        grid=(grid,),
        in_specs=[pl.BlockSpec((ROWS_PER_BLOCK, neurons), lambda i: (i, 0))],
        out_specs=pl.BlockSpec((ROWS_PER_BLOCK, neurons), lambda i: (i, 0)),
        out_shape=jax.ShapeDtypeStruct(s.shape, s.dtype),
    )(s)
